# baseline (speedup 1.0000x reference)
import functools

import jax
import jax.numpy as jnp
from jax import lax
from jax.experimental import pallas as pl
from jax.experimental.pallas import tpu as pltpu
from jax.experimental.pallas import tpu_sc as plsc

S = 2048
D = 1024
E = 64
DFF = 1024
DFFS = 1024
T = 128
ALIGN = 8
S_PAD = 2816

_SC_NC = 2
_SC_NS = 16
_NW = _SC_NC * _SC_NS
L = 16

_TPW = S // _SC_NS
_PPW = S_PAD // _SC_NS
_NG = _TPW // L


def _silu(x):
    return x * jax.nn.sigmoid(x)


def _sc_route(logits_flat):
    mesh = plsc.VectorSubcoreMesh(
        core_axis_name="c", subcore_axis_name="s", num_cores=1)

    @functools.partial(
        pl.kernel,
        out_type=(
            jax.ShapeDtypeStruct((E,), jnp.int32),
            jax.ShapeDtypeStruct((E,), jnp.int32),
            jax.ShapeDtypeStruct((S,), jnp.int32),
            jax.ShapeDtypeStruct((S_PAD,), jnp.int32),
            jax.ShapeDtypeStruct((S_PAD,), jnp.float32),
        ),
        mesh=mesh,
        compiler_params=pltpu.CompilerParams(needs_layout_passes=False),
        scratch_types=[
            pltpu.VMEM((E * _TPW,), jnp.float32),
            pltpu.VMEM((_TPW,), jnp.int32),
            pltpu.VMEM((_TPW,), jnp.float32),
            pltpu.VMEM((_TPW,), jnp.int32),
            pltpu.VMEM((E,), jnp.int32),
            pltpu.VMEM((_SC_NS * E,), jnp.int32),
            pltpu.VMEM((E,), jnp.int32),
            pltpu.VMEM((E,), jnp.int32),
            pltpu.VMEM((E,), jnp.int32),
            pltpu.VMEM((_TPW,), jnp.int32),
            pltpu.VMEM((_TPW,), jnp.int32),
            pltpu.VMEM((_PPW,), jnp.int32),
            pltpu.VMEM((_PPW,), jnp.float32),
            pltpu.VMEM_SHARED((_SC_NS * 512,), jnp.int32),
            pltpu.SemaphoreType.DMA,
        ],
    )
    def route_kernel(logits_hbm, counts_hbm, starts_hbm, invpos_hbm,
                     perm_hbm, wpad_hbm, ltile, selv, wv, rankv, hist,
                     allhist, cntv, stv, sbv, posb, tokb, zi, zf, sh_hist,
                     sem):
        wid = lax.axis_index("s")
        tbase = wid * _TPW

        for e_ in range(E):
            pltpu.sync_copy(
                logits_hbm.at[pl.ds(e_ * S + tbase, _TPW)],
                ltile.at[pl.ds(e_ * _TPW, _TPW)])

        iota = lax.iota(jnp.int32, L)

        def bc_i(s):
            return jnp.broadcast_to(s, (L,))

        def bc_f(s):
            return jnp.broadcast_to(s, (L,))

        def a1_init():
            ms, ss, sels = [], [], []
            for g in range(_NG):
                lv = ltile[pl.ds(g * L, L)]
                ms.append(lv)
                ss.append(jnp.full((L,), 1.0, jnp.float32))
                sels.append(jnp.zeros((L,), jnp.int32))
            return tuple(ms + ss + sels)

        def a1_body(e, carry):
            ms = list(carry[:_NG])
            ss = list(carry[_NG:2 * _NG])
            sels = list(carry[2 * _NG:])
            for g in range(_NG):
                lv = ltile[pl.ds(e * _TPW + g * L, L)]
                upd = lv > ms[g]
                mnew = jnp.maximum(ms[g], lv)
                ss[g] = (ss[g] * jnp.exp(ms[g] - mnew)
                         + jnp.exp(lv - mnew))
                ms[g] = mnew
                sels[g] = jnp.where(upd, bc_i(e), sels[g])
            return tuple(ms + ss + sels)

        a1 = lax.fori_loop(1, E, a1_body, a1_init())
        one = jnp.full((L,), 1.0, jnp.float32)
        for g in range(_NG):
            selv[pl.ds(g * L, L)] = a1[2 * _NG + g]
            wv[pl.ds(g * L, L)] = one / a1[_NG + g]

        pbase = wid * _PPW
        for k in range(_PPW // L):
            zi[pl.ds(k * L, L)] = (bc_i(pbase + k * L) + iota) & bc_i(
                jnp.int32(S - 1))
            zf[pl.ds(k * L, L)] = jnp.zeros((L,), jnp.float32)
        pltpu.sync_copy(zi, perm_hbm.at[pl.ds(pbase, _PPW)])
        pltpu.sync_copy(zf, wpad_hbm.at[pl.ds(pbase, _PPW)])

        for k in range(_TPW // L):
            rankv[pl.ds(k * L, L)] = jnp.zeros((L,), jnp.int32)

        def bin_body(e, carry):
            tot = jnp.int32(0)
            eb = bc_i(e)
            for k in range(_TPW // L):
                sv = selv[pl.ds(k * L, L)]
                me = sv == eb
                mk = me.astype(jnp.int32)
                ck = plsc.cumsum(mk)
                rk = rankv[pl.ds(k * L, L)]
                rankv[pl.ds(k * L, L)] = jnp.where(
                    me, bc_i(tot) + ck - bc_i(jnp.int32(1)), rk)
                tot = tot + jnp.sum(mk)
            grp = e // L
            lane = e - grp * L
            hv = hist[pl.ds(grp * L, L)]
            hist[pl.ds(grp * L, L)] = jnp.where(
                iota == bc_i(lane), bc_i(tot), hv)
            return carry

        lax.fori_loop(0, E, bin_body, 0)

        pltpu.sync_copy(hist, sh_hist.at[pl.ds(wid * 512, E)])
        plsc.subcore_barrier()
        for w in range(_SC_NS):
            pltpu.sync_copy(sh_hist.at[pl.ds(w * 512, E)],
                            allhist.at[pl.ds(w * E, E)])

        zero_v = jnp.zeros((L,), jnp.int32)
        carry = jnp.int32(0)
        for g in range(E // L):
            cnt_g = jnp.zeros((L,), jnp.int32)
            base_g = jnp.zeros((L,), jnp.int32)
            for w in range(_SC_NS):
                v = allhist[pl.ds(w * E + g * L, L)]
                cnt_g = cnt_g + v
                base_g = base_g + jnp.where(
                    bc_i(jnp.int32(w)) < bc_i(wid), v, zero_v)
            seg_g = ((cnt_g + bc_i(jnp.int32(ALIGN - 1))) >> 3) << 3
            cs_g = plsc.cumsum(seg_g)
            st_g = cs_g - seg_g + bc_i(carry)
            carry = carry + jnp.sum(seg_g)
            cntv[pl.ds(g * L, L)] = cnt_g
            stv[pl.ds(g * L, L)] = st_g
            sbv[pl.ds(g * L, L)] = st_g + base_g

        @pl.when(wid == 0)
        def _():
            pltpu.sync_copy(cntv, counts_hbm)
            pltpu.sync_copy(stv, starts_hbm)

        for g in range(_NG):
            sv = selv[pl.ds(g * L, L)]
            rk = rankv[pl.ds(g * L, L)]
            sb = plsc.load_gather(sbv, [sv])
            posb[pl.ds(g * L, L)] = sb + rk
            tokb[pl.ds(g * L, L)] = bc_i(tbase + g * L) + iota

        pltpu.sync_copy(posb, invpos_hbm.at[pl.ds(tbase, _TPW)])
        plsc.subcore_barrier()
        pltpu.async_copy(tokb, perm_hbm.at[posb], sem).wait()
        pltpu.async_copy(wv, wpad_hbm.at[posb], sem).wait()

    return route_kernel(logits_flat)


def _sc_row_gather(table, idx):
    n_rows = idx.shape[0]
    d = table.shape[1]
    b_per_w = n_rows // _NW
    mesh = plsc.VectorSubcoreMesh(core_axis_name="c", subcore_axis_name="s")

    @functools.partial(
        pl.kernel,
        out_type=jax.ShapeDtypeStruct((n_rows, d), table.dtype),
        mesh=mesh,
        scratch_types=[
            pltpu.VMEM((b_per_w,), jnp.int32),
            pltpu.VMEM((b_per_w, d), table.dtype),
            pltpu.SemaphoreType.DMA,
        ],
    )
    def gather_kernel(table_hbm, idx_hbm, out_hbm, idx_v, rows_v, sem):
        wid = lax.axis_index("s") * _SC_NC + lax.axis_index("c")
        base = wid * b_per_w
        pltpu.sync_copy(idx_hbm.at[pl.ds(base, b_per_w)], idx_v)
        pltpu.async_copy(table_hbm.at[idx_v], rows_v, sem).wait()
        pltpu.sync_copy(rows_v, out_hbm.at[pl.ds(base, b_per_w)])

    return gather_kernel(table, idx)


def _moe_ffn_body(starts_ref, counts_ref, xs_ref, w_ref, gw_ref, uw_ref,
                  dw_ref, out_ref):
    e = pl.program_id(0)
    start = starts_ref[e]
    cnt = counts_ref[e]
    n_tiles = (cnt + T - 1) // T

    def tile_body(t, carry):
        r = pl.multiple_of(start + t * T, ALIGN)
        xt = xs_ref[pl.ds(r, T), :]
        g = lax.dot_general(xt, gw_ref[0], (((1,), (1,)), ((), ())))
        u = lax.dot_general(xt, uw_ref[0], (((1,), (1,)), ((), ())))
        h = _silu(g) * u * w_ref[pl.ds(r, T), :]
        o = lax.dot_general(h, dw_ref[0], (((1,), (1,)), ((), ())))
        row = r + lax.broadcasted_iota(jnp.int32, (T, 1), 0)
        keep = row < start + cnt
        out_ref[pl.ds(r, T), :] = jnp.where(keep, o, out_ref[pl.ds(r, T), :])
        return carry

    lax.fori_loop(0, n_tiles, tile_body, 0)


def _moe_ffn(xs, w_pad, starts, counts, expert_gate_w, expert_up_w,
             expert_down_w):
    grid_spec = pltpu.PrefetchScalarGridSpec(
        num_scalar_prefetch=2,
        grid=(E,),
        in_specs=[
            pl.BlockSpec(memory_space=pltpu.MemorySpace.VMEM),
            pl.BlockSpec(memory_space=pltpu.MemorySpace.VMEM),
            pl.BlockSpec((1, DFF, D), lambda e, s, c: (e, 0, 0)),
            pl.BlockSpec((1, DFF, D), lambda e, s, c: (e, 0, 0)),
            pl.BlockSpec((1, D, DFF), lambda e, s, c: (e, 0, 0)),
        ],
        out_specs=pl.BlockSpec(memory_space=pltpu.MemorySpace.VMEM),
    )
    return pl.pallas_call(
        _moe_ffn_body,
        grid_spec=grid_spec,
        out_shape=jax.ShapeDtypeStruct((S_PAD, D), jnp.float32),
    )(starts, counts, xs, w_pad, expert_gate_w, expert_up_w, expert_down_w)


_TS = 256


def _shared_body(x_ref, moe_ref, sg_ref, su_ref, sd_ref, seg_ref, out_ref):
    xt = x_ref[...]
    g = lax.dot_general(xt, sg_ref[...], (((1,), (1,)), ((), ())))
    u = lax.dot_general(xt, su_ref[...], (((1,), (1,)), ((), ())))
    h = _silu(g) * u
    y = lax.dot_general(h, sd_ref[...], (((1,), (1,)), ((), ())))
    gate = jax.nn.sigmoid(
        lax.dot_general(xt, seg_ref[...], (((1,), (1,)), ((), ()))))
    out_ref[...] = y * gate + moe_ref[...]


def _shared_ffn(x, moe, shared_gate_w, shared_up_w, shared_down_w,
                shared_expert_gate_w):
    return pl.pallas_call(
        _shared_body,
        grid=(S // _TS,),
        in_specs=[
            pl.BlockSpec((_TS, D), lambda i: (i, 0)),
            pl.BlockSpec((_TS, D), lambda i: (i, 0)),
            pl.BlockSpec(memory_space=pltpu.MemorySpace.VMEM),
            pl.BlockSpec(memory_space=pltpu.MemorySpace.VMEM),
            pl.BlockSpec(memory_space=pltpu.MemorySpace.VMEM),
            pl.BlockSpec(memory_space=pltpu.MemorySpace.VMEM),
        ],
        out_specs=pl.BlockSpec((_TS, D), lambda i: (i, 0)),
        out_shape=jax.ShapeDtypeStruct((S, D), jnp.float32),
    )(x, moe, shared_gate_w, shared_up_w, shared_down_w, shared_expert_gate_w)


def kernel(hidden_states, gate_w, expert_gate_w, expert_up_w, expert_down_w,
           shared_gate_w, shared_up_w, shared_down_w, shared_expert_gate_w):
    b, s, d = hidden_states.shape
    x = hidden_states.reshape(-1, d)

    router_logits = x @ gate_w.T

    counts, starts, inv_pos, perm_padded, w_pad = _sc_route(
        router_logits.T.reshape(-1))
    w_pad = w_pad.reshape(S_PAD, 1)

    xs = _sc_row_gather(x, perm_padded)

    out_sorted = _moe_ffn(xs, w_pad, starts, counts,
                          expert_gate_w, expert_up_w, expert_down_w)

    moe = _sc_row_gather(out_sorted, inv_pos)

    final = _shared_ffn(x, moe, shared_gate_w, shared_up_w, shared_down_w,
                        shared_expert_gate_w)
    return final.reshape(b, s, d)


# speedup vs baseline: 1.0759x; 1.0759x over previous
import functools

import jax
import jax.numpy as jnp
from jax import lax
from jax.experimental import pallas as pl
from jax.experimental.pallas import tpu as pltpu
from jax.experimental.pallas import tpu_sc as plsc

S = 2048
D = 1024
E = 64
DFF = 1024
DFFS = 1024
T = 128
ALIGN = 8
S_PAD = 2816

_SC_NC = 2
_SC_NS = 16
_NW = _SC_NC * _SC_NS
L = 16

_TPW = S // _SC_NS
_PPW = S_PAD // _SC_NS
_NG = _TPW // L


def _silu(x):
    return x * jax.nn.sigmoid(x)


def _sc_route(logits_flat):
    mesh = plsc.VectorSubcoreMesh(
        core_axis_name="c", subcore_axis_name="s", num_cores=1)

    @functools.partial(
        pl.kernel,
        out_type=(
            jax.ShapeDtypeStruct((E,), jnp.int32),
            jax.ShapeDtypeStruct((E,), jnp.int32),
            jax.ShapeDtypeStruct((S,), jnp.int32),
            jax.ShapeDtypeStruct((S_PAD,), jnp.int32),
            jax.ShapeDtypeStruct((S_PAD,), jnp.float32),
        ),
        mesh=mesh,
        compiler_params=pltpu.CompilerParams(needs_layout_passes=False),
        scratch_types=[
            pltpu.VMEM((_TPW * E,), jnp.float32),
            pltpu.VMEM((_TPW,), jnp.int32),
            pltpu.VMEM((_TPW,), jnp.float32),
            pltpu.VMEM((_TPW,), jnp.int32),
            pltpu.VMEM((E,), jnp.int32),
            pltpu.VMEM((_SC_NS * E,), jnp.int32),
            pltpu.VMEM((E,), jnp.int32),
            pltpu.VMEM((E,), jnp.int32),
            pltpu.VMEM((E,), jnp.int32),
            pltpu.VMEM((_TPW,), jnp.int32),
            pltpu.VMEM((_TPW,), jnp.int32),
            pltpu.VMEM((_PPW,), jnp.int32),
            pltpu.VMEM((_PPW,), jnp.float32),
            pltpu.VMEM_SHARED((_SC_NS * 512,), jnp.int32),
            pltpu.SemaphoreType.DMA,
        ],
    )
    def route_kernel(logits_hbm, counts_hbm, starts_hbm, invpos_hbm,
                     perm_hbm, wpad_hbm, ltile, selv, wv, rankv, hist,
                     allhist, cntv, stv, sbv, posb, tokb, zi, zf, sh_hist,
                     sem):
        wid = lax.axis_index("s")
        tbase = wid * _TPW

        pltpu.sync_copy(logits_hbm.at[pl.ds(tbase * E, _TPW * E)], ltile)

        iota = lax.iota(jnp.int32, L)

        def bc_i(s):
            return jnp.broadcast_to(s, (L,))

        def bc_f(s):
            return jnp.broadcast_to(s, (L,))

        def tok_group(g, carry):
            sel_acc = jnp.zeros((L,), jnp.int32)
            w_acc = jnp.zeros((L,), jnp.float32)
            for j in range(L):
                t = g * L + j
                l0 = ltile[pl.ds(t * E, L)]
                l1 = ltile[pl.ds(t * E + L, L)]
                l2 = ltile[pl.ds(t * E + 2 * L, L)]
                l3 = ltile[pl.ds(t * E + 3 * L, L)]
                m = jnp.max(jnp.maximum(jnp.maximum(l0, l1),
                                        jnp.maximum(l2, l3)))
                mb = bc_f(m)
                ssum = (jnp.sum(jnp.exp(l0 - mb)) + jnp.sum(jnp.exp(l1 - mb))
                        + jnp.sum(jnp.exp(l2 - mb))
                        + jnp.sum(jnp.exp(l3 - mb)))
                big = bc_i(jnp.int32(4 * L))
                i0 = jnp.min(jnp.where(l0 == mb, iota, big))
                i1 = jnp.min(jnp.where(l1 == mb, iota + L, big))
                i2 = jnp.min(jnp.where(l2 == mb, iota + 2 * L, big))
                i3 = jnp.min(jnp.where(l3 == mb, iota + 3 * L, big))
                sel_t = jnp.minimum(jnp.minimum(i0, i1), jnp.minimum(i2, i3))
                w_vec = jnp.full((L,), 1.0, jnp.float32) / bc_f(ssum)
                lane_m = iota == bc_i(jnp.int32(j))
                sel_acc = jnp.where(lane_m, bc_i(sel_t), sel_acc)
                w_acc = jnp.where(lane_m, w_vec, w_acc)
            selv[pl.ds(g * L, L)] = sel_acc
            wv[pl.ds(g * L, L)] = w_acc
            return carry

        lax.fori_loop(0, _NG, tok_group, 0)

        pbase = wid * _PPW
        for k in range(_PPW // L):
            zi[pl.ds(k * L, L)] = (bc_i(pbase + k * L) + iota) & bc_i(
                jnp.int32(S - 1))
            zf[pl.ds(k * L, L)] = jnp.zeros((L,), jnp.float32)
        pltpu.sync_copy(zi, perm_hbm.at[pl.ds(pbase, _PPW)])
        pltpu.sync_copy(zf, wpad_hbm.at[pl.ds(pbase, _PPW)])

        for k in range(_TPW // L):
            rankv[pl.ds(k * L, L)] = jnp.zeros((L,), jnp.int32)

        def bin_body(e, carry):
            tot = jnp.int32(0)
            eb = bc_i(e)
            for k in range(_TPW // L):
                sv = selv[pl.ds(k * L, L)]
                me = sv == eb
                mk = me.astype(jnp.int32)
                ck = plsc.cumsum(mk)
                rk = rankv[pl.ds(k * L, L)]
                rankv[pl.ds(k * L, L)] = jnp.where(
                    me, bc_i(tot) + ck - bc_i(jnp.int32(1)), rk)
                tot = tot + jnp.sum(mk)
            grp = e // L
            lane = e - grp * L
            hv = hist[pl.ds(grp * L, L)]
            hist[pl.ds(grp * L, L)] = jnp.where(
                iota == bc_i(lane), bc_i(tot), hv)
            return carry

        lax.fori_loop(0, E, bin_body, 0)

        pltpu.sync_copy(hist, sh_hist.at[pl.ds(wid * 512, E)])
        plsc.subcore_barrier()
        for w in range(_SC_NS):
            pltpu.sync_copy(sh_hist.at[pl.ds(w * 512, E)],
                            allhist.at[pl.ds(w * E, E)])

        zero_v = jnp.zeros((L,), jnp.int32)
        carry = jnp.int32(0)
        for g in range(E // L):
            cnt_g = jnp.zeros((L,), jnp.int32)
            base_g = jnp.zeros((L,), jnp.int32)
            for w in range(_SC_NS):
                v = allhist[pl.ds(w * E + g * L, L)]
                cnt_g = cnt_g + v
                base_g = base_g + jnp.where(
                    bc_i(jnp.int32(w)) < bc_i(wid), v, zero_v)
            seg_g = ((cnt_g + bc_i(jnp.int32(ALIGN - 1))) >> 3) << 3
            cs_g = plsc.cumsum(seg_g)
            st_g = cs_g - seg_g + bc_i(carry)
            carry = carry + jnp.sum(seg_g)
            cntv[pl.ds(g * L, L)] = cnt_g
            stv[pl.ds(g * L, L)] = st_g
            sbv[pl.ds(g * L, L)] = st_g + base_g

        @pl.when(wid == 0)
        def _():
            pltpu.sync_copy(cntv, counts_hbm)
            pltpu.sync_copy(stv, starts_hbm)

        for g in range(_NG):
            sv = selv[pl.ds(g * L, L)]
            rk = rankv[pl.ds(g * L, L)]
            sb = plsc.load_gather(sbv, [sv])
            posb[pl.ds(g * L, L)] = sb + rk
            tokb[pl.ds(g * L, L)] = bc_i(tbase + g * L) + iota

        pltpu.sync_copy(posb, invpos_hbm.at[pl.ds(tbase, _TPW)])
        plsc.subcore_barrier()
        pltpu.async_copy(tokb, perm_hbm.at[posb], sem).wait()
        pltpu.async_copy(wv, wpad_hbm.at[posb], sem).wait()

    return route_kernel(logits_flat)


def _sc_row_gather(table, idx):
    n_rows = idx.shape[0]
    d = table.shape[1]
    b_per_w = n_rows // _NW
    mesh = plsc.VectorSubcoreMesh(core_axis_name="c", subcore_axis_name="s")

    @functools.partial(
        pl.kernel,
        out_type=jax.ShapeDtypeStruct((n_rows, d), table.dtype),
        mesh=mesh,
        scratch_types=[
            pltpu.VMEM((b_per_w,), jnp.int32),
            pltpu.VMEM((b_per_w, d), table.dtype),
            pltpu.SemaphoreType.DMA,
        ],
    )
    def gather_kernel(table_hbm, idx_hbm, out_hbm, idx_v, rows_v, sem):
        wid = lax.axis_index("s") * _SC_NC + lax.axis_index("c")
        base = wid * b_per_w
        pltpu.sync_copy(idx_hbm.at[pl.ds(base, b_per_w)], idx_v)
        pltpu.async_copy(table_hbm.at[idx_v], rows_v, sem).wait()
        pltpu.sync_copy(rows_v, out_hbm.at[pl.ds(base, b_per_w)])

    return gather_kernel(table, idx)


def _moe_ffn_body(starts_ref, counts_ref, xs_ref, w_ref, gw_ref, uw_ref,
                  dw_ref, out_ref):
    e = pl.program_id(0)
    start = starts_ref[e]
    cnt = counts_ref[e]
    n_tiles = (cnt + T - 1) // T

    def tile_body(t, carry):
        r = pl.multiple_of(start + t * T, ALIGN)
        xt = xs_ref[pl.ds(r, T), :]
        g = lax.dot_general(xt, gw_ref[0], (((1,), (1,)), ((), ())))
        u = lax.dot_general(xt, uw_ref[0], (((1,), (1,)), ((), ())))
        h = _silu(g) * u * w_ref[pl.ds(r, T), :]
        o = lax.dot_general(h, dw_ref[0], (((1,), (1,)), ((), ())))
        row = r + lax.broadcasted_iota(jnp.int32, (T, 1), 0)
        keep = row < start + cnt
        out_ref[pl.ds(r, T), :] = jnp.where(keep, o, out_ref[pl.ds(r, T), :])
        return carry

    lax.fori_loop(0, n_tiles, tile_body, 0)


def _moe_ffn(xs, w_pad, starts, counts, expert_gate_w, expert_up_w,
             expert_down_w):
    grid_spec = pltpu.PrefetchScalarGridSpec(
        num_scalar_prefetch=2,
        grid=(E,),
        in_specs=[
            pl.BlockSpec(memory_space=pltpu.MemorySpace.VMEM),
            pl.BlockSpec(memory_space=pltpu.MemorySpace.VMEM),
            pl.BlockSpec((1, DFF, D), lambda e, s, c: (e, 0, 0)),
            pl.BlockSpec((1, DFF, D), lambda e, s, c: (e, 0, 0)),
            pl.BlockSpec((1, D, DFF), lambda e, s, c: (e, 0, 0)),
        ],
        out_specs=pl.BlockSpec(memory_space=pltpu.MemorySpace.VMEM),
    )
    return pl.pallas_call(
        _moe_ffn_body,
        grid_spec=grid_spec,
        out_shape=jax.ShapeDtypeStruct((S_PAD, D), jnp.float32),
    )(starts, counts, xs, w_pad, expert_gate_w, expert_up_w, expert_down_w)


_TS = 256


def _shared_body(x_ref, moe_ref, sg_ref, su_ref, sd_ref, seg_ref, out_ref):
    xt = x_ref[...]
    g = lax.dot_general(xt, sg_ref[...], (((1,), (1,)), ((), ())))
    u = lax.dot_general(xt, su_ref[...], (((1,), (1,)), ((), ())))
    h = _silu(g) * u
    y = lax.dot_general(h, sd_ref[...], (((1,), (1,)), ((), ())))
    gate = jax.nn.sigmoid(
        lax.dot_general(xt, seg_ref[...], (((1,), (1,)), ((), ()))))
    out_ref[...] = y * gate + moe_ref[...]


def _shared_ffn(x, moe, shared_gate_w, shared_up_w, shared_down_w,
                shared_expert_gate_w):
    return pl.pallas_call(
        _shared_body,
        grid=(S // _TS,),
        in_specs=[
            pl.BlockSpec((_TS, D), lambda i: (i, 0)),
            pl.BlockSpec((_TS, D), lambda i: (i, 0)),
            pl.BlockSpec(memory_space=pltpu.MemorySpace.VMEM),
            pl.BlockSpec(memory_space=pltpu.MemorySpace.VMEM),
            pl.BlockSpec(memory_space=pltpu.MemorySpace.VMEM),
            pl.BlockSpec(memory_space=pltpu.MemorySpace.VMEM),
        ],
        out_specs=pl.BlockSpec((_TS, D), lambda i: (i, 0)),
        out_shape=jax.ShapeDtypeStruct((S, D), jnp.float32),
    )(x, moe, shared_gate_w, shared_up_w, shared_down_w, shared_expert_gate_w)


def kernel(hidden_states, gate_w, expert_gate_w, expert_up_w, expert_down_w,
           shared_gate_w, shared_up_w, shared_down_w, shared_expert_gate_w):
    b, s, d = hidden_states.shape
    x = hidden_states.reshape(-1, d)

    router_logits = x @ gate_w.T

    counts, starts, inv_pos, perm_padded, w_pad = _sc_route(
        router_logits.reshape(-1))
    w_pad = w_pad.reshape(S_PAD, 1)

    xs = _sc_row_gather(x, perm_padded)

    out_sorted = _moe_ffn(xs, w_pad, starts, counts,
                          expert_gate_w, expert_up_w, expert_down_w)

    moe = _sc_row_gather(out_sorted, inv_pos)

    final = _shared_ffn(x, moe, shared_gate_w, shared_up_w, shared_down_w,
                        shared_expert_gate_w)
    return final.reshape(b, s, d)
